# Initial kernel scaffold; baseline (speedup 1.0000x reference)
import jax
import jax.numpy as jnp
from jax.experimental import pallas as pl


def kernel(token_ids, token_table, pos_table):
    raise NotImplementedError("write your pallas kernel here")


# speedup vs baseline: 2.4765x; 2.4765x over previous
import jax
import jax.numpy as jnp
from jax import lax
from jax.experimental import pallas as pl
from jax.experimental.pallas import tpu as pltpu
from jax.experimental.pallas import tpu_sc as plsc

DIM = 64
B = 4096
S = 200
NC = 2
NS = 16
NW = NC * NS
TOT = B * S
CHUNK = 400
SUB = 100
NSUB = CHUNK // SUB
NCHUNKS = TOT // CHUNK
CPW = NCHUNKS // NW


def _emb_body(ids_hbm, table_hbm, pos2_hbm, out_hbm, idx_v, pos_v, rows_v, sem):
    cid = lax.axis_index("c")
    sid = lax.axis_index("s")
    wid = sid * NC + cid

    pltpu.sync_copy(pos2_hbm, pos_v)

    @pl.loop(0, CPW)
    def _chunk_loop(i):
        chunk = wid * CPW + i
        pltpu.sync_copy(ids_hbm.at[chunk], idx_v)
        copies = [
            pltpu.async_copy(
                table_hbm.at[idx_v.at[j]],
                rows_v.at[pl.ds(j * SUB, SUB)],
                sem,
            )
            for j in range(NSUB)
        ]
        for c in copies:
            c.wait()

        @pl.loop(0, CHUNK, unroll=8)
        def _add_loop(r):
            for c4 in range(DIM // 16):
                plsc.addupdate(
                    rows_v.at[r, pl.ds(c4 * 16, 16)],
                    pos_v[r, pl.ds(c4 * 16, 16)],
                )

        pltpu.sync_copy(rows_v, out_hbm.at[chunk])


def kernel(token_ids, token_table, pos_table):
    ids = token_ids.astype(jnp.int32).reshape(NCHUNKS, NSUB, SUB)
    pos2 = jnp.concatenate([pos_table[:S], pos_table[:S]], axis=0)
    k = pl.kernel(
        _emb_body,
        out_type=jax.ShapeDtypeStruct((NCHUNKS, CHUNK, DIM), jnp.float32),
        mesh=plsc.VectorSubcoreMesh(core_axis_name="c", subcore_axis_name="s"),
        compiler_params=pltpu.CompilerParams(use_tc_tiling_on_sc=False),
        scratch_types=[
            pltpu.VMEM((NSUB, SUB), jnp.int32),
            pltpu.VMEM((CHUNK, DIM), jnp.float32),
            pltpu.VMEM((CHUNK, DIM), jnp.float32),
            pltpu.SemaphoreType.DMA,
        ],
    )
    out = k(ids, token_table, pos2)
    return out.reshape(B, S, DIM)


# speedup vs baseline: 2.7591x; 1.1141x over previous
import jax
import jax.numpy as jnp
from jax import lax
from jax.experimental import pallas as pl
from jax.experimental.pallas import tpu as pltpu
from jax.experimental.pallas import tpu_sc as plsc

DIM = 64
B = 4096
S = 200
NC = 2
NS = 16
NW = NC * NS
TOT = B * S
CHUNK = 400
SUB = 100
NSUB = CHUNK // SUB
NCHUNKS = TOT // CHUNK
CPW = NCHUNKS // NW


def _emb_body(ids_hbm, table_hbm, pos2_hbm, out_hbm,
              idx_all, pos_v, rows0, rows1, sem0, sem1):
    cid = lax.axis_index("c")
    sid = lax.axis_index("s")
    wid = sid * NC + cid

    pltpu.sync_copy(ids_hbm.at[wid], idx_all)
    pltpu.sync_copy(pos2_hbm, pos_v)

    def gather_into(i, rows, sem):
        for j in range(NSUB):
            pltpu.async_copy(
                table_hbm.at[idx_all.at[i * NSUB + j]],
                rows.at[pl.ds(j * SUB, SUB)],
                sem,
            )

    def wait_gather(rows, sem):
        for j in range(NSUB):
            pltpu.make_async_copy(
                table_hbm.at[idx_all.at[j]],
                rows.at[pl.ds(j * SUB, SUB)],
                sem,
            ).wait()

    def add_pos_and_flush(i, rows):
        @pl.loop(0, CHUNK, unroll=8)
        def _add_loop(r):
            for c4 in range(DIM // 16):
                plsc.addupdate(
                    rows.at[r, pl.ds(c4 * 16, 16)],
                    pos_v[r, pl.ds(c4 * 16, 16)],
                )
        pltpu.sync_copy(rows, out_hbm.at[wid * CPW + i])

    gather_into(0, rows0, sem0)

    @pl.loop(0, CPW, step=2)
    def _chunk_loop(i):
        wait_gather(rows0, sem0)
        gather_into(i + 1, rows1, sem1)
        add_pos_and_flush(i, rows0)

        wait_gather(rows1, sem1)

        @pl.when(i + 2 < CPW)
        def _():
            gather_into(i + 2, rows0, sem0)

        add_pos_and_flush(i + 1, rows1)


def kernel(token_ids, token_table, pos_table):
    ids = token_ids.astype(jnp.int32).reshape(NW, CPW * NSUB, SUB)
    pos2 = jnp.concatenate([pos_table[:S], pos_table[:S]], axis=0)
    k = pl.kernel(
        _emb_body,
        out_type=jax.ShapeDtypeStruct((NCHUNKS, CHUNK, DIM), jnp.float32),
        mesh=plsc.VectorSubcoreMesh(core_axis_name="c", subcore_axis_name="s"),
        compiler_params=pltpu.CompilerParams(use_tc_tiling_on_sc=False),
        scratch_types=[
            pltpu.VMEM((CPW * NSUB, SUB), jnp.int32),
            pltpu.VMEM((CHUNK, DIM), jnp.float32),
            pltpu.VMEM((CHUNK, DIM), jnp.float32),
            pltpu.VMEM((CHUNK, DIM), jnp.float32),
            pltpu.SemaphoreType.DMA,
            pltpu.SemaphoreType.DMA,
        ],
    )
    out = k(ids, token_table, pos2)
    return out.reshape(B, S, DIM)


# speedup vs baseline: 2.7667x; 1.0028x over previous
import jax
import jax.numpy as jnp
from jax import lax
from jax.experimental import pallas as pl
from jax.experimental.pallas import tpu as pltpu
from jax.experimental.pallas import tpu_sc as plsc

DIM = 64
B = 4096
S = 200
NC = 2
NS = 16
NW = NC * NS
RPW = B // NW
CHUNK = 2
SUB = 100
NSUB = (CHUNK * S) // SUB
CPW = RPW // CHUNK


def _emb_body(ids_hbm, table_hbm, pos_hbm, out_hbm,
              idx_all, pos_v, rows0, rows1, sem0, sem1):
    cid = lax.axis_index("c")
    sid = lax.axis_index("s")
    wid = sid * NC + cid
    row0 = wid * RPW

    pltpu.sync_copy(ids_hbm.at[pl.ds(row0, RPW)], idx_all)
    pltpu.sync_copy(pos_hbm.at[pl.ds(0, S)], pos_v.at[0])
    pltpu.sync_copy(pos_hbm.at[pl.ds(0, S)], pos_v.at[1])

    def gather_into(i, rows, sem):
        for b in range(CHUNK):
            pltpu.async_copy(
                table_hbm.at[idx_all.at[CHUNK * i + b]],
                rows.at[b],
                sem,
            )

    def wait_gather(rows, sem):
        for b in range(CHUNK):
            pltpu.make_async_copy(
                table_hbm.at[idx_all.at[b]],
                rows.at[b],
                sem,
            ).wait()

    def add_pos_and_flush(i, rows):
        for b in range(CHUNK):
            @pl.loop(0, S, unroll=8)
            def _add_loop(r):
                for c4 in range(DIM // 16):
                    plsc.addupdate(
                        rows.at[b, r, pl.ds(c4 * 16, 16)],
                        pos_v[b, r, pl.ds(c4 * 16, 16)],
                    )
        pltpu.sync_copy(rows, out_hbm.at[pl.ds(row0 + CHUNK * i, CHUNK)])

    gather_into(0, rows0, sem0)

    @pl.loop(0, CPW, step=2)
    def _chunk_loop(i):
        wait_gather(rows0, sem0)
        gather_into(i + 1, rows1, sem1)
        add_pos_and_flush(i, rows0)

        wait_gather(rows1, sem1)

        @pl.when(i + 2 < CPW)
        def _():
            gather_into(i + 2, rows0, sem0)

        add_pos_and_flush(i + 1, rows1)


def kernel(token_ids, token_table, pos_table):
    k = pl.kernel(
        _emb_body,
        out_type=jax.ShapeDtypeStruct((B, S, DIM), jnp.float32),
        mesh=plsc.VectorSubcoreMesh(core_axis_name="c", subcore_axis_name="s"),
        compiler_params=pltpu.CompilerParams(use_tc_tiling_on_sc=False),
        scratch_types=[
            pltpu.VMEM((RPW, S), jnp.int32),
            pltpu.VMEM((CHUNK, S, DIM), jnp.float32),
            pltpu.VMEM((CHUNK, S, DIM), jnp.float32),
            pltpu.VMEM((CHUNK, S, DIM), jnp.float32),
            pltpu.SemaphoreType.DMA,
            pltpu.SemaphoreType.DMA,
        ],
    )
    return k(token_ids.astype(jnp.int32), token_table, pos_table)
